# baseline (speedup 1.0000x reference)
import dataclasses
import functools
import math

import jax
import jax.numpy as jnp
from jax import lax
from jax.experimental import pallas as pl
from jax.experimental.pallas import tpu as pltpu
from jax.experimental.pallas import tpu_sc as plsc

N = 4096
SIZE = 32000
SMOOTHING = 0.1
CONFIDENCE = 1.0 - SMOOTHING
FILL = SMOOTHING / (SIZE - 1)
K_CONST = N * (SIZE - 1) * FILL * math.log(FILL) + N * CONFIDENCE * math.log(CONFIDENCE)

NC = 2
NW = 32
BPW = N // NW
CHUNK = 16

NSTREAM = 16
SLAB = 8
STEP_ROWS = NSTREAM * SLAB


def _sc_pick_body(x_hbm, tgt_hbm, out_hbm, tgt_v, val2d, out_v, sem):
    wid = lax.axis_index("s") * NC + lax.axis_index("c")
    base = wid * BPW
    pltpu.sync_copy(tgt_hbm.at[pl.ds(base, BPW)], tgt_v)

    @pl.loop(0, BPW, step=CHUNK)
    def _(k):
        tv = tgt_v[pl.ds(k, CHUNK)]
        copies = []
        for u in range(CHUNK):
            t = tv[u]
            t8 = pl.multiple_of((t // 8) * 8, 8)
            copies.append(
                pltpu.async_copy(
                    x_hbm.at[base + k + u, pl.ds(t8, 8)],
                    val2d.at[k + u],
                    sem,
                )
            )
        for c in copies:
            c.wait()

    @pl.loop(0, BPW, step=16)
    def _(k):
        rows = lax.iota(jnp.int32, 16) + k
        rem = tgt_v[pl.ds(k, 16)] & 7
        out_v[pl.ds(k, 16)] = plsc.load_gather(val2d, [rows, rem])

    pltpu.sync_copy(out_v, out_hbm.at[pl.ds(base, BPW)])


@functools.lru_cache(maxsize=1)
def _sc_pick():
    cp = pltpu.CompilerParams()
    if "needs_layout_passes" in pltpu.CompilerParams.__dataclass_fields__:
        cp = dataclasses.replace(cp, needs_layout_passes=False)
    return pl.kernel(
        _sc_pick_body,
        out_type=jax.ShapeDtypeStruct((N,), jnp.float32),
        compiler_params=cp,
        mesh=plsc.VectorSubcoreMesh(core_axis_name="c", subcore_axis_name="s"),
        scratch_types=[
            pltpu.VMEM((BPW,), jnp.int32),
            pltpu.VMEM((BPW, 8), jnp.float32),
            pltpu.VMEM((BPW,), jnp.float32),
            pltpu.SemaphoreType.DMA,
        ],
    )


def _main_body(*refs):
    x_refs, s_ref = refs[:NSTREAM], refs[NSTREAM]
    i = pl.program_id(0)

    s = jnp.float32(0.0)
    for g in range(NSTREAM // 4):
        p = (
            x_refs[4 * g][...]
            * x_refs[4 * g + 1][...]
            * x_refs[4 * g + 2][...]
            * x_refs[4 * g + 3][...]
        )
        s += jnp.sum(jnp.log(p))

    @pl.when(i == 0)
    def _():
        s_ref[...] = jnp.zeros_like(s_ref)

    s_ref[...] += s


def _combine_body(g_ref, s_ref, o_ref):
    s_tgt = jnp.sum(jnp.log(g_ref[...]))
    o_ref[...] = K_CONST - FILL * s_ref[...] - (CONFIDENCE - FILL) * s_tgt


def kernel(x, target):
    picked = _sc_pick()(x, target)

    s_all = pl.pallas_call(
        _main_body,
        grid=(N // STEP_ROWS,),
        in_specs=[
            pl.BlockSpec((SLAB, SIZE), (lambda i, j=j: (i * NSTREAM + j, 0)))
            for j in range(NSTREAM)
        ],
        out_specs=pl.BlockSpec((1, 1), lambda i: (0, 0)),
        out_shape=jax.ShapeDtypeStruct((1, 1), jnp.float32),
    )(*([x] * NSTREAM))

    loss = pl.pallas_call(
        _combine_body,
        in_specs=[
            pl.BlockSpec((NW, BPW), lambda: (0, 0)),
            pl.BlockSpec((1, 1), lambda: (0, 0)),
        ],
        out_specs=pl.BlockSpec((1, 1), lambda: (0, 0)),
        out_shape=jax.ShapeDtypeStruct((1, 1), jnp.float32),
    )(picked.reshape(NW, BPW), s_all)

    return loss.reshape(())


# speedup vs baseline: 1.0022x; 1.0022x over previous
import dataclasses
import functools
import math

import jax
import jax.numpy as jnp
from jax import lax
from jax.experimental import pallas as pl
from jax.experimental.pallas import tpu as pltpu
from jax.experimental.pallas import tpu_sc as plsc

N = 4096
SIZE = 32000
SMOOTHING = 0.1
CONFIDENCE = 1.0 - SMOOTHING
FILL = SMOOTHING / (SIZE - 1)
K_CONST = N * (SIZE - 1) * FILL * math.log(FILL) + N * CONFIDENCE * math.log(CONFIDENCE)

NC = 2
NW = 32
BPW = N // NW
CHUNK = 16

NSTREAM = 16
SLAB = 8
STEP_ROWS = NSTREAM * SLAB


def _sc_pick_body(x_hbm, tgt_hbm, out_hbm, tgt_v, val2d, out_v, sem):
    wid = lax.axis_index("s") * NC + lax.axis_index("c")
    base = wid * BPW
    pltpu.sync_copy(tgt_hbm.at[pl.ds(base, BPW)], tgt_v)

    pending = []
    for k in range(0, BPW, CHUNK):
        tv = tgt_v[pl.ds(k, CHUNK)]
        copies = []
        for u in range(CHUNK):
            t = tv[u]
            t8 = pl.multiple_of((t // 8) * 8, 8)
            copies.append(
                pltpu.async_copy(
                    x_hbm.at[base + k + u, pl.ds(t8, 8)],
                    val2d.at[k + u],
                    sem,
                )
            )
        for c in pending:
            c.wait()
        pending = copies
    for c in pending:
        c.wait()

    @pl.loop(0, BPW, step=16)
    def _(k):
        rows = lax.iota(jnp.int32, 16) + k
        rem = tgt_v[pl.ds(k, 16)] & 7
        out_v[pl.ds(k, 16)] = plsc.load_gather(val2d, [rows, rem])

    pltpu.sync_copy(out_v, out_hbm.at[pl.ds(base, BPW)])


@functools.lru_cache(maxsize=1)
def _sc_pick():
    cp = pltpu.CompilerParams()
    if "needs_layout_passes" in pltpu.CompilerParams.__dataclass_fields__:
        cp = dataclasses.replace(cp, needs_layout_passes=False)
    return pl.kernel(
        _sc_pick_body,
        out_type=jax.ShapeDtypeStruct((N,), jnp.float32),
        compiler_params=cp,
        mesh=plsc.VectorSubcoreMesh(core_axis_name="c", subcore_axis_name="s"),
        scratch_types=[
            pltpu.VMEM((BPW,), jnp.int32),
            pltpu.VMEM((BPW, 8), jnp.float32),
            pltpu.VMEM((BPW,), jnp.float32),
            pltpu.SemaphoreType.DMA,
        ],
    )


def _main_body(*refs):
    x_refs, s_ref = refs[:NSTREAM], refs[NSTREAM]
    i = pl.program_id(0)

    s = jnp.float32(0.0)
    for g in range(NSTREAM // 4):
        p = (
            x_refs[4 * g][...]
            * x_refs[4 * g + 1][...]
            * x_refs[4 * g + 2][...]
            * x_refs[4 * g + 3][...]
        )
        s += jnp.sum(jnp.log(p))

    @pl.when(i == 0)
    def _():
        s_ref[...] = jnp.zeros_like(s_ref)

    s_ref[...] += s


def _combine_body(g_ref, s_ref, o_ref):
    s_tgt = jnp.sum(jnp.log(g_ref[...]))
    o_ref[...] = K_CONST - FILL * s_ref[...] - (CONFIDENCE - FILL) * s_tgt


def kernel(x, target):
    picked = _sc_pick()(x, target)

    s_all = pl.pallas_call(
        _main_body,
        grid=(N // STEP_ROWS,),
        in_specs=[
            pl.BlockSpec((SLAB, SIZE), (lambda i, j=j: (i * NSTREAM + j, 0)))
            for j in range(NSTREAM)
        ],
        out_specs=pl.BlockSpec((1, 1), lambda i: (0, 0)),
        out_shape=jax.ShapeDtypeStruct((1, 1), jnp.float32),
    )(*([x] * NSTREAM))

    loss = pl.pallas_call(
        _combine_body,
        in_specs=[
            pl.BlockSpec((NW, BPW), lambda: (0, 0)),
            pl.BlockSpec((1, 1), lambda: (0, 0)),
        ],
        out_specs=pl.BlockSpec((1, 1), lambda: (0, 0)),
        out_shape=jax.ShapeDtypeStruct((1, 1), jnp.float32),
    )(picked.reshape(NW, BPW), s_all)

    return loss.reshape(())
